# baseline (speedup 1.0000x reference)
import jax
import jax.numpy as jnp
from jax.experimental import pallas as pl
from jax.experimental.pallas import tpu as pltpu

HID = 1024
NE = 8
INTER = 2752
T = 4096
TOPK = 2

BM = 256
NB = 40
PADT = NB * BM
BI = 256
IB = (INTER + BI - 1) // BI
LAST_VALID = INTER - (IB - 1) * BI


def _routing_body(x_ref, wg_ref, sel_ref, wts_ref):
    logits = jnp.dot(x_ref[...], wg_ref[...],
                     preferred_element_type=jnp.float32)
    eids = jax.lax.broadcasted_iota(jnp.int32, logits.shape, 1)
    m1 = jnp.max(logits, axis=1, keepdims=True)
    e1 = jnp.min(jnp.where(logits == m1, eids, NE), axis=1, keepdims=True)
    l2m = jnp.where(eids == e1, -jnp.inf, logits)
    m2 = jnp.max(l2m, axis=1, keepdims=True)
    e2 = jnp.min(jnp.where(l2m == m2, eids, NE), axis=1, keepdims=True)
    wa = jax.lax.logistic(m1 - m2)
    sel_ref[...] = jnp.concatenate([e1, e2], axis=1)
    wts_ref[...] = jnp.concatenate([wa, 1.0 - wa], axis=1)


def _ffn_body(be_ref, xg_ref, w1_ref, w3_ref, w2_ref, ws_ref, h_ref):
    ib = pl.program_id(0)
    nb = pl.program_id(1)
    x = xg_ref[...]
    w1 = w1_ref[0].astype(jnp.bfloat16)
    w3 = w3_ref[0].astype(jnp.bfloat16)
    a = jnp.dot(x, w1, preferred_element_type=jnp.float32)
    b = jnp.dot(x, w3, preferred_element_type=jnp.float32)
    g = a * jax.lax.logistic(a) * b
    valid = jnp.where(ib == IB - 1, LAST_VALID, BI)
    gcol = jax.lax.broadcasted_iota(jnp.int32, g.shape, 1)
    g = jnp.where(gcol < valid, g, 0.0)
    w2 = w2_ref[0]
    wrow = jax.lax.broadcasted_iota(jnp.int32, w2.shape, 0)
    w2 = jnp.where(wrow < valid, w2, 0.0).astype(jnp.bfloat16)
    h = jnp.dot(g.astype(jnp.bfloat16), w2,
                preferred_element_type=jnp.float32)
    rows = pl.ds(nb * BM, BM)

    @pl.when(ib == 0)
    def _():
        h_ref[rows, :] = h

    @pl.when(ib > 0)
    def _():
        h_ref[rows, :] += h

    @pl.when(ib == IB - 1)
    def _():
        h_ref[rows, :] *= ws_ref[0]


def kernel(hidden_states, Wg, W1, W2, W3):
    bs, seq, hid = hidden_states.shape
    x = hidden_states.reshape(-1, hid)

    sel, wts = pl.pallas_call(
        _routing_body,
        grid=(1,),
        in_specs=[
            pl.BlockSpec((T, HID), lambda i: (0, 0)),
            pl.BlockSpec((HID, NE), lambda i: (0, 0)),
        ],
        out_specs=[
            pl.BlockSpec((T, TOPK), lambda i: (0, 0)),
            pl.BlockSpec((T, TOPK), lambda i: (0, 0)),
        ],
        out_shape=[
            jax.ShapeDtypeStruct((T, TOPK), jnp.int32),
            jax.ShapeDtypeStruct((T, TOPK), jnp.float32),
        ],
    )(x, Wg)

    fe = sel.reshape(-1)
    oh = (fe[:, None] == jnp.arange(NE)[None, :]).astype(jnp.int32)
    csum = jnp.cumsum(oh, axis=0)
    rank = jnp.sum((csum - oh) * oh, axis=1)
    counts = csum[-1]
    seg = -(-counts // BM) * BM
    astart = jnp.concatenate(
        [jnp.zeros((1,), jnp.int32), jnp.cumsum(seg)])[:NE]
    pos = astart[fe] + rank
    tok = jnp.arange(T * TOPK, dtype=jnp.int32) // TOPK
    src = jnp.zeros((PADT,), jnp.int32).at[pos].set(tok)
    wrow = jnp.zeros((PADT,), jnp.float32).at[pos].set(wts.reshape(-1))
    block_rows = jnp.arange(NB, dtype=jnp.int32) * BM
    be = (jnp.sum(astart[None, :] <= block_rows[:, None], axis=1)
          .astype(jnp.int32) - 1)

    xg = x[src].astype(jnp.bfloat16)
    ws3 = wrow.reshape(NB, BM, 1)

    h = pl.pallas_call(
        _ffn_body,
        grid_spec=pltpu.PrefetchScalarGridSpec(
            num_scalar_prefetch=1,
            grid=(IB, NB),
            in_specs=[
                pl.BlockSpec((BM, HID), lambda ib, nb, be_s: (nb, 0)),
                pl.BlockSpec((1, HID, BI), lambda ib, nb, be_s: (be_s[nb], 0, ib)),
                pl.BlockSpec((1, HID, BI), lambda ib, nb, be_s: (be_s[nb], 0, ib)),
                pl.BlockSpec((1, BI, HID), lambda ib, nb, be_s: (be_s[nb], ib, 0)),
                pl.BlockSpec((1, BM, 1), lambda ib, nb, be_s: (nb, 0, 0)),
            ],
            out_specs=pl.BlockSpec((PADT, HID), lambda ib, nb, be_s: (0, 0)),
        ),
        out_shape=jax.ShapeDtypeStruct((PADT, HID), jnp.float32),
        compiler_params=pltpu.CompilerParams(
            dimension_semantics=("arbitrary", "arbitrary"),
        ),
    )(be, xg, W1, W3, W2, ws3)

    p = pos.reshape(T, TOPK)
    out = h[p[:, 0]] + h[p[:, 1]]
    return out.reshape(bs, seq, hid)
